# Initial kernel scaffold; baseline (speedup 1.0000x reference)
import jax
import jax.numpy as jnp
from jax.experimental import pallas as pl


def kernel(event_id, url_id, url_table, event_table, W, b):
    raise NotImplementedError("write your pallas kernel here")


# speedup vs baseline: 1.4051x; 1.4051x over previous
import functools

import jax
import jax.numpy as jnp
from jax import lax
from jax.experimental import pallas as pl
from jax.experimental.pallas import tpu as pltpu
from jax.experimental.pallas import tpu_sc as plsc

NUM_URL = 1000000
URL_EMB_DIM = 64
NUM_EVENT = 1000
EVENT_EMB_DIM = 64
ITEM_EMB_DIM = 128
BATCH = 16384
HIST = 50

N = BATCH * HIST
CHUNK = 128
NCHUNK = N // CHUNK
NWORK = 32
CPW = NCHUNK // NWORK


def _sc_gather_kernel(url_table, event_table, uidx, eidx, uout, eout,
                      uidx_v, eidx_v, ubuf, ebuf, usem, esem):
    wid = lax.axis_index("s") * 2 + lax.axis_index("c")
    base = wid * CPW
    pltpu.sync_copy(uidx.at[pl.ds(base, CPW)], uidx_v)
    pltpu.sync_copy(eidx.at[pl.ds(base, CPW)], eidx_v)

    def body(i, _):
        cu = pltpu.async_copy(url_table.at[uidx_v.at[i]], ubuf, usem)
        ce = pltpu.async_copy(event_table.at[eidx_v.at[i]], ebuf, esem)
        cu.wait()
        ce.wait()
        pltpu.sync_copy(ubuf, uout.at[base + i])
        pltpu.sync_copy(ebuf, eout.at[base + i])
        return 0

    lax.fori_loop(0, CPW, body, 0)


def _sc_gather(url_table, event_table, uidx, eidx):
    mesh = plsc.VectorSubcoreMesh(core_axis_name="c", subcore_axis_name="s")
    f = functools.partial(
        pl.kernel,
        out_type=(
            jax.ShapeDtypeStruct((NCHUNK, CHUNK, URL_EMB_DIM), jnp.float32),
            jax.ShapeDtypeStruct((NCHUNK, CHUNK, EVENT_EMB_DIM), jnp.float32),
        ),
        mesh=mesh,
        compiler_params=pltpu.CompilerParams(use_tc_tiling_on_sc=False),
        scratch_types=[
            pltpu.VMEM((CPW, CHUNK), jnp.int32),
            pltpu.VMEM((CPW, CHUNK), jnp.int32),
            pltpu.VMEM((CHUNK, URL_EMB_DIM), jnp.float32),
            pltpu.VMEM((CHUNK, EVENT_EMB_DIM), jnp.float32),
            pltpu.SemaphoreType.DMA,
            pltpu.SemaphoreType.DMA,
        ],
    )(_sc_gather_kernel)
    return f(url_table, event_table, uidx, eidx)


def _tc_matmul_kernel(uref, eref, wref, bref, oref):
    u = uref[...]
    e = eref[...]
    w = wref[...]
    acc = lax.dot_general(u, w[:, :URL_EMB_DIM],
                          dimension_numbers=(((1,), (1,)), ((), ())),
                          preferred_element_type=jnp.float32)
    acc = acc + lax.dot_general(e, w[:, URL_EMB_DIM:],
                                dimension_numbers=(((1,), (1,)), ((), ())),
                                preferred_element_type=jnp.float32)
    oref[...] = jnp.maximum(acc + bref[...], 0.0)


def _tc_matmul(url_g, ev_g, W, b):
    TB = 2048
    grid = (N // TB,)
    return pl.pallas_call(
        _tc_matmul_kernel,
        grid=grid,
        in_specs=[
            pl.BlockSpec((TB, URL_EMB_DIM), lambda i: (i, 0)),
            pl.BlockSpec((TB, EVENT_EMB_DIM), lambda i: (i, 0)),
            pl.BlockSpec((ITEM_EMB_DIM, URL_EMB_DIM + EVENT_EMB_DIM),
                         lambda i: (0, 0)),
            pl.BlockSpec((1, ITEM_EMB_DIM), lambda i: (0, 0)),
        ],
        out_specs=pl.BlockSpec((TB, ITEM_EMB_DIM), lambda i: (i, 0)),
        out_shape=jax.ShapeDtypeStruct((N, ITEM_EMB_DIM), jnp.float32),
    )(url_g, ev_g, W, b)


def kernel(event_id, url_id, url_table, event_table, W, b):
    uidx = url_id.reshape(NCHUNK, CHUNK).astype(jnp.int32)
    eidx = event_id.reshape(NCHUNK, CHUNK).astype(jnp.int32)
    url_g, ev_g = _sc_gather(url_table, event_table, uidx, eidx)
    out = _tc_matmul(url_g.reshape(N, URL_EMB_DIM),
                     ev_g.reshape(N, EVENT_EMB_DIM),
                     W, b.reshape(1, ITEM_EMB_DIM))
    return out.reshape(BATCH, HIST, ITEM_EMB_DIM)


# speedup vs baseline: 4.1864x; 2.9795x over previous
import functools

import jax
import jax.numpy as jnp
from jax import lax
from jax.experimental import pallas as pl
from jax.experimental.pallas import tpu as pltpu
from jax.experimental.pallas import tpu_sc as plsc

NUM_URL = 1000000
URL_EMB_DIM = 64
NUM_EVENT = 1000
EVENT_EMB_DIM = 64
CAT_DIM = URL_EMB_DIM + EVENT_EMB_DIM
ITEM_EMB_DIM = 128
BATCH = 16384
HIST = 50

N = BATCH * HIST
CHUNK = 128
NCHUNK = N // CHUNK
NWORK = 32
SLICES = 5
HS = HIST // SLICES
NCHUNK_S = NCHUNK // SLICES
NS = N // SLICES
CPW = NCHUNK_S // NWORK
NBUF = 4
STEPS = CPW // NBUF


def _sc_gather_kernel(url_table, event_table, uidx, eidx, cout,
                      uidx_v, eidx_v, ubuf, ebuf, *sems):
    gsem = sems[:NBUF]
    wsem = sems[NBUF:]
    wid = lax.axis_index("s") * 2 + lax.axis_index("c")
    base = wid * CPW
    pltpu.sync_copy(uidx.at[pl.ds(base, CPW)], uidx_v)
    pltpu.sync_copy(eidx.at[pl.ds(base, CPW)], eidx_v)

    def gather_descs(s, c):
        return (pltpu.make_async_copy(url_table.at[uidx_v.at[c]],
                                      ubuf.at[s], gsem[s]),
                pltpu.make_async_copy(event_table.at[eidx_v.at[c]],
                                      ebuf.at[s], gsem[s]))

    def write_descs(s, c):
        row0 = (base + c) * CHUNK
        return (pltpu.make_async_copy(
                    ubuf.at[s],
                    cout.at[pl.ds(row0, CHUNK), pl.ds(0, URL_EMB_DIM)],
                    wsem[s]),
                pltpu.make_async_copy(
                    ebuf.at[s],
                    cout.at[pl.ds(row0, CHUNK), pl.ds(URL_EMB_DIM, EVENT_EMB_DIM)],
                    wsem[s]))

    def body(j, _):
        c0 = j * NBUF
        for s in range(NBUF):
            @pl.when(j > 0)
            def _(s=s, c0=c0):
                for d in write_descs(s, c0 + s):
                    d.wait()
            for d in gather_descs(s, c0 + s):
                d.start()
        for s in range(NBUF):
            for d in gather_descs(s, c0 + s):
                d.wait()
            for d in write_descs(s, c0 + s):
                d.start()
        return 0

    lax.fori_loop(0, STEPS, body, 0)
    for s in range(NBUF):
        for d in write_descs(s, (STEPS - 1) * NBUF + s):
            d.wait()


def _sc_gather(url_table, event_table, uidx, eidx):
    mesh = plsc.VectorSubcoreMesh(core_axis_name="c", subcore_axis_name="s")
    f = functools.partial(
        pl.kernel,
        out_type=jax.ShapeDtypeStruct((NS, CAT_DIM), jnp.float32),
        mesh=mesh,
        compiler_params=pltpu.CompilerParams(use_tc_tiling_on_sc=False),
        scratch_types=(
            [pltpu.VMEM((CPW, CHUNK), jnp.int32),
             pltpu.VMEM((CPW, CHUNK), jnp.int32),
             pltpu.VMEM((NBUF, CHUNK, URL_EMB_DIM), jnp.float32),
             pltpu.VMEM((NBUF, CHUNK, EVENT_EMB_DIM), jnp.float32)]
            + [pltpu.SemaphoreType.DMA] * (2 * NBUF)
        ),
    )(_sc_gather_kernel)
    return f(url_table, event_table, uidx, eidx)


PAIR = 512000
TBC = 4096
NBLK = PAIR // TBC
VALID2 = (NUM_URL - TBC) // TBC
TAIL = 1024
PROWS = PAIR + TAIL // 2


def _tc_transpose_kernel(t1ref, t2ref, oref):
    oref[...] = jnp.concatenate([t1ref[...].T, t2ref[...].T], axis=1)


def _tc_tail_kernel(t3ref, prev_ref, oref):
    del prev_ref
    t3 = t3ref[...]
    oref[...] = jnp.concatenate([t3[:, :TAIL // 2].T, t3[:, TAIL // 2:].T],
                                axis=1)


def _tc_transpose(table_t, tail_t):
    main = pl.pallas_call(
        _tc_transpose_kernel,
        grid=(NBLK,),
        in_specs=[
            pl.BlockSpec((URL_EMB_DIM, TBC), lambda i: (0, i)),
            pl.BlockSpec((URL_EMB_DIM, TBC),
                         lambda i: (0, jnp.minimum(i + NBLK, VALID2))),
        ],
        out_specs=pl.BlockSpec((TBC, 2 * URL_EMB_DIM), lambda i: (i, 0)),
        out_shape=jax.ShapeDtypeStruct((PROWS, 2 * URL_EMB_DIM),
                                       jnp.float32),
    )(table_t, table_t)
    return pl.pallas_call(
        _tc_tail_kernel,
        grid=(1,),
        in_specs=[
            pl.BlockSpec((URL_EMB_DIM, TAIL), lambda i: (0, 0)),
            pl.BlockSpec(memory_space=pltpu.MemorySpace.HBM),
        ],
        out_specs=pl.BlockSpec((TAIL // 2, 2 * URL_EMB_DIM),
                               lambda i: (PAIR // (TAIL // 2), 0)),
        out_shape=jax.ShapeDtypeStruct((PROWS, 2 * URL_EMB_DIM),
                                       jnp.float32),
        input_output_aliases={1: 0},
    )(tail_t, main)


def _tc_matmul_kernel(cref, wref, bref, oref):
    c = cref[...]
    w = wref[...]
    acc = lax.dot_general(c, w, dimension_numbers=(((1,), (1,)), ((), ())),
                          preferred_element_type=jnp.float32)
    out = jnp.maximum(acc + bref[...], 0.0)
    oref[...] = out.reshape(oref.shape)


def _tc_matmul_tail_kernel(cref, wref, bref, pref, oref):
    del pref
    _tc_matmul_kernel(cref, wref, bref, oref)


def _tc_matmul_slice(cat_s, W, b, prev, s):
    BB = 4096
    JB = BATCH // BB
    grid = (HS, JB)
    out_shape = jax.ShapeDtypeStruct((HIST, BATCH, ITEM_EMB_DIM),
                                     jnp.float32)
    in_specs = [
        pl.BlockSpec((BB, CAT_DIM), lambda h, j: (h * JB + j, 0)),
        pl.BlockSpec((ITEM_EMB_DIM, CAT_DIM), lambda h, j: (0, 0)),
        pl.BlockSpec((1, ITEM_EMB_DIM), lambda h, j: (0, 0)),
    ]
    out_spec = pl.BlockSpec((1, BB, ITEM_EMB_DIM),
                            lambda h, j, s=s: (s * HS + h, j, 0))
    if prev is None:
        return pl.pallas_call(
            _tc_matmul_kernel, grid=grid, in_specs=in_specs,
            out_specs=out_spec, out_shape=out_shape,
        )(cat_s, W, b)
    return pl.pallas_call(
        _tc_matmul_tail_kernel, grid=grid,
        in_specs=in_specs + [pl.BlockSpec(memory_space=pltpu.MemorySpace.HBM)],
        out_specs=out_spec, out_shape=out_shape,
        input_output_aliases={3: 0},
    )(cat_s, W, b, prev)


def kernel(event_id, url_id, url_table, event_table, W, b):
    u = url_id.T.reshape(NCHUNK, CHUNK).astype(jnp.int32)
    edge2 = (VALID2 + 1) * TBC
    t0 = NUM_URL - TAIL
    uidx = jnp.where(
        u < PAIR, 2 * u,
        jnp.where(u < edge2, 2 * (u - PAIR) + 1,
                  jnp.where(u < t0 + TAIL // 2,
                            2 * (PAIR + u - t0),
                            2 * (PAIR + u - t0 - TAIL // 2) + 1)))
    eidx = event_id.T.reshape(NCHUNK, CHUNK).astype(jnp.int32)
    tail_t = url_table[NUM_URL - TAIL:].T
    url_lin = _tc_transpose(url_table.T, tail_t).reshape(2 * PROWS,
                                                         URL_EMB_DIM)
    b2 = b.reshape(1, ITEM_EMB_DIM)
    out = None
    for s in range(SLICES):
        lo = s * NCHUNK_S
        cat_s = _sc_gather(url_lin, event_table,
                           uidx[lo:lo + NCHUNK_S], eidx[lo:lo + NCHUNK_S])
        out = _tc_matmul_slice(cat_s, W, b2, out, s)
    return out.transpose(1, 0, 2)
